# baseline (speedup 1.0000x reference)
import jax
import jax.numpy as jnp
import numpy as np
from jax.experimental import pallas as pl

_INDEX_ORDER = np.array([
    [0, 1, 5, 6, 14, 15, 27, 28],
    [2, 4, 7, 13, 16, 26, 29, 42],
    [3, 8, 12, 17, 25, 30, 41, 43],
    [9, 11, 18, 24, 31, 40, 44, 53],
    [10, 19, 23, 32, 39, 45, 52, 54],
    [20, 22, 33, 38, 46, 51, 55, 60],
    [21, 34, 37, 47, 50, 56, 59, 61],
    [35, 36, 48, 49, 57, 58, 62, 63]], dtype=np.int32).flatten()

_P64 = np.zeros((64, 64), dtype=np.float32)
_P64[_INDEX_ORDER, np.arange(64)] = 1.0
_P128 = np.zeros((128, 128), dtype=np.float32)
_P128[:64, :64] = _P64
_P128[64:, 64:] = _P64


def _permute_block(x_ref, p_ref, o_ref):
    o_ref[...] = jnp.dot(x_ref[...], p_ref[...],
                         preferred_element_type=jnp.float32,
                         precision=jax.lax.Precision.HIGHEST)


def kernel(x):
    B, num_blocks = x.shape[0], x.shape[1]
    n = B * num_blocks * 64 // 128
    x2 = x.reshape(n, 128)
    rows_blk = min(2048, n)
    out = pl.pallas_call(
        _permute_block,
        grid=(n // rows_blk,),
        in_specs=[
            pl.BlockSpec((rows_blk, 128), lambda i: (i, 0)),
            pl.BlockSpec((128, 128), lambda i: (0, 0)),
        ],
        out_specs=pl.BlockSpec((rows_blk, 128), lambda i: (i, 0)),
        out_shape=jax.ShapeDtypeStruct((n, 128), jnp.float32),
    )(x2, jnp.asarray(_P128))
    return out.reshape(B, num_blocks, 64)
